# baseline (speedup 1.0000x reference)
import functools

import jax
import jax.numpy as jnp
from jax import lax
from jax.experimental import pallas as pl
from jax.experimental.pallas import tpu as pltpu
from jax.experimental.pallas import tpu_sc as plsc

N1 = 20000
N2 = 4096
D = 128
DH = D // 2
NC = 2
NS = 16
NW = NC * NS
BLK = 128
CH = 16


CNT_R = 4224


def _seg_sum_body(src_hbm, dst_hbm, dstc_hbm, table_hbm, acc_hbm, cnt_hbm,
                  src_v, dst_v, dstc_v, rows0, rows1, ones_v, zc_v,
                  acc_sh, cnt_sh, g0, g1, s0, s1, *, ew, accr):
    c = lax.axis_index("c")
    s = lax.axis_index("s")
    slab = accr // NS
    cslab = CNT_R // NS
    rows_v = rows0
    bufs = (rows0, rows1)
    gsems = (g0, g1)
    ssems = (s0, s1)

    zv = jnp.zeros((16,), jnp.float32)
    ov = jnp.ones((16,), jnp.float32)

    def zrows(i, carry):
        r = i // (DH // 16)
        col = (i % (DH // 16)) * 16
        rows_v[r, pl.ds(col, 16)] = zv
        return carry
    lax.fori_loop(0, BLK * (DH // 16), zrows, 0)

    def zfill(i, carry):
        ones_v[i % BLK, pl.ds(0, 16)] = ov
        zc_v[i, pl.ds(0, 16)] = zv
        return carry
    lax.fori_loop(0, BLK, zfill, 0)

    base = s * slab
    cbase = s * cslab
    def zacc(i, carry):
        r = base + i * BLK
        nrow = jnp.minimum(slab - i * BLK, BLK)
        @pl.when(nrow == BLK)
        def _():
            pltpu.sync_copy(rows_v, acc_sh.at[pl.ds(r, BLK)])
        @pl.when(nrow < BLK)
        def _():
            pltpu.sync_copy(rows_v.at[pl.ds(0, slab % BLK)],
                            acc_sh.at[pl.ds(r, slab % BLK)])
        return carry
    lax.fori_loop(0, -(-slab // BLK), zacc, 0)
    def zcnt(i, carry):
        r = cbase + i * BLK
        nrow = jnp.minimum(cslab - i * BLK, BLK)
        @pl.when(nrow == BLK)
        def _():
            pltpu.sync_copy(zc_v, cnt_sh.at[pl.ds(r, BLK)])
        @pl.when(nrow < BLK)
        def _():
            pltpu.sync_copy(zc_v.at[pl.ds(0, cslab % BLK)],
                            cnt_sh.at[pl.ds(r, cslab % BLK)])
        return carry
    lax.fori_loop(0, -(-cslab // BLK), zcnt, 0)
    plsc.subcore_barrier()

    nblk = ew // BLK
    def chunk(t, carry):
        row0 = s * nblk + t * CH
        pltpu.sync_copy(src_hbm.at[pl.ds(c * (NS * nblk) + row0, CH)], src_v)
        pltpu.sync_copy(dst_hbm.at[pl.ds(row0, CH)], dst_v)
        pltpu.sync_copy(dstc_hbm.at[pl.ds(row0, CH)], dstc_v)
        gcp = {0: pltpu.async_copy(table_hbm.at[src_v.at[0]], bufs[0],
                                   gsems[0])}
        scp = {}
        for j in range(CH):
            if j + 1 < CH:
                if j - 1 in scp:
                    scp.pop(j - 1).wait()
                gcp[j + 1] = pltpu.async_copy(
                    table_hbm.at[src_v.at[j + 1]], bufs[(j + 1) % 2],
                    gsems[(j + 1) % 2])
            gcp.pop(j).wait()
            scp[j] = pltpu.async_copy(bufs[j % 2], acc_sh.at[dst_v.at[j]],
                                      ssems[j % 2], add=True)
            @pl.when(lax.rem(jnp.int32(j), 2) == c)
            def _():
                pltpu.sync_copy(ones_v, cnt_sh.at[dstc_v.at[j]], add=True)
        for cp in scp.values():
            cp.wait()
        return carry
    lax.fori_loop(0, nblk // CH, chunk, 0)

    plsc.subcore_barrier()
    pltpu.sync_copy(acc_sh.at[pl.ds(base, slab)],
                    acc_hbm.at[c, pl.ds(base, slab)])
    pltpu.sync_copy(cnt_sh.at[pl.ds(cbase, cslab)],
                    cnt_hbm.at[c, pl.ds(cbase, cslab)])


@functools.lru_cache(maxsize=None)
def _make_seg_kernel(ew, accr):
    mesh = plsc.VectorSubcoreMesh(core_axis_name="c", subcore_axis_name="s")
    return pl.kernel(
        functools.partial(_seg_sum_body, ew=ew, accr=accr),
        out_type=[jax.ShapeDtypeStruct((NC, accr, DH), jnp.float32),
                  jax.ShapeDtypeStruct((NC, CNT_R, 16), jnp.float32)],
        mesh=mesh,
        compiler_params=pltpu.CompilerParams(use_tc_tiling_on_sc=False),
        scratch_types=[
            pltpu.VMEM((CH, BLK), jnp.int32),
            pltpu.VMEM((CH, BLK), jnp.int32),
            pltpu.VMEM((CH, BLK), jnp.int32),
            pltpu.VMEM((BLK, DH), jnp.float32),
            pltpu.VMEM((BLK, DH), jnp.float32),
            pltpu.VMEM((BLK, 16), jnp.float32),
            pltpu.VMEM((BLK, 16), jnp.float32),
            pltpu.VMEM_SHARED((accr, DH), jnp.float32),
            pltpu.VMEM_SHARED((CNT_R, 16), jnp.float32),
            pltpu.SemaphoreType.DMA,
            pltpu.SemaphoreType.DMA,
            pltpu.SemaphoreType.DMA,
            pltpu.SemaphoreType.DMA,
        ],
    )


def _epilogue_body(acc_ref, cnt_ref, xt_ref, wl_ref, b_ref, wr_ref, o_ref,
                   *, act):
    ab = acc_ref[...]
    cb = cnt_ref[...]
    a = jnp.concatenate([ab[0], ab[1]], axis=-1)
    cnt = cb[0, :, 0:1] + cb[1, :, 0:1]
    agg = a / jnp.clip(cnt, 1.0, None)
    xt = xt_ref[...]
    if xt.ndim == 3:
        xt = jnp.concatenate([xt[0], xt[1]], axis=-1)
    z = (jnp.dot(agg, wl_ref[...], preferred_element_type=jnp.float32)
         + b_ref[...]
         + jnp.dot(xt, wr_ref[...], preferred_element_type=jnp.float32))
    if act == "relu":
        h = jnp.maximum(z, 0.0)
        o_ref[0] = h[:, :DH]
        o_ref[1] = h[:, DH:]
    else:
        mx = jnp.max(z, axis=-1, keepdims=True)
        e = jnp.exp(z - mx)
        o_ref[...] = z - mx - jnp.log(jnp.sum(e, axis=-1, keepdims=True))


@functools.lru_cache(maxsize=None)
def _make_epilogue(act, rows=512):
    grid = (N2 // rows,)
    if act == "relu":
        xt_spec = pl.BlockSpec((rows, D), lambda i: (i, 0))
        out_spec = pl.BlockSpec((NC, rows, DH), lambda i: (0, i, 0))
        out_shape = jax.ShapeDtypeStruct((NC, N2, DH), jnp.float32)
    else:
        xt_spec = pl.BlockSpec((NC, rows, DH), lambda i: (0, i, 0))
        out_spec = pl.BlockSpec((rows, D), lambda i: (i, 0))
        out_shape = jax.ShapeDtypeStruct((N2, D), jnp.float32)
    return pl.pallas_call(
        functools.partial(_epilogue_body, act=act),
        grid=grid,
        in_specs=[
            pl.BlockSpec((NC, rows, DH), lambda i: (0, i, 0)),
            pl.BlockSpec((NC, rows, 16), lambda i: (0, i, 0)),
            xt_spec,
            pl.BlockSpec((D, D), lambda i: (0, 0)),
            pl.BlockSpec((1, D), lambda i: (0, 0)),
            pl.BlockSpec((D, D), lambda i: (0, 0)),
        ],
        out_specs=out_spec,
        out_shape=out_shape,
    )


def _pad_edges(src, dst, garbage, ngarb):
    e = src.shape[0]
    ew = -(-e // (NS * CH * BLK)) * CH * BLK
    pad = NS * ew - e
    if pad:
        r = jnp.arange(pad, dtype=jnp.int32)
        src = jnp.concatenate([src, r % 1024])
        dst = jnp.concatenate([dst, garbage + r % ngarb])
    return src, dst, ew


def kernel(x, edge_index0, edge_index1, size0_dst, size1_dst,
           W0_l, b0_l, W0_r, W1_l, b1_l, W1_r):
    x = x.astype(jnp.float32)
    acc_r0 = N1 + 96
    acc_r1 = N2 + 128
    src0, dst0, ew0 = _pad_edges(edge_index0[0].astype(jnp.int32),
                                 edge_index0[1].astype(jnp.int32), N1, 96)
    src1, dst1, ew1 = _pad_edges(edge_index1[0].astype(jnp.int32),
                                 edge_index1[1].astype(jnp.int32), N2, 128)
    z0 = jnp.asarray(size0_dst - N1, dtype=x.dtype)
    z1 = jnp.asarray(size1_dst - N2, dtype=x.dtype)

    n0 = x.shape[0]
    x_h = jnp.concatenate([x[:, :DH], x[:, DH:]])
    src0x = jnp.concatenate([src0, src0 + n0]).reshape(-1, BLK)
    dstc0 = jnp.where(dst0 < N2, dst0, N2 + lax.rem(dst0, 128)).reshape(-1, BLK)
    dst0 = dst0.reshape(-1, BLK)
    acc0, cnt0 = _make_seg_kernel(ew0, acc_r0)(src0x, dst0, dstc0, x_h)
    xt = x[:N2] + z0
    h_h = _make_epilogue("relu")(acc0[:, :N2], cnt0[:, :N2], xt,
                                 W0_l, b0_l.reshape(1, D), W0_r)
    ht_h = h_h + z1
    ht_flat = ht_h.reshape(NC * N2, DH)
    src1x = jnp.concatenate([src1, src1 + N2]).reshape(-1, BLK)
    dstc1 = jnp.where(dst1 < N2, dst1, N2 + lax.rem(dst1, 128)).reshape(-1, BLK)
    dst1 = dst1.reshape(-1, BLK)
    acc1, cnt1 = _make_seg_kernel(ew1, acc_r1)(src1x, dst1, dstc1, ht_flat)
    out = _make_epilogue("lsm")(acc1[:, :N2], cnt1[:, :N2], ht_h,
                                W1_l, b1_l.reshape(1, D), W1_r)
    return out


# speedup vs baseline: 1.0127x; 1.0127x over previous
import functools

import jax
import jax.numpy as jnp
from jax import lax
from jax.experimental import pallas as pl
from jax.experimental.pallas import tpu as pltpu
from jax.experimental.pallas import tpu_sc as plsc

N1 = 20000
N2 = 4096
D = 128
DH = D // 2
NC = 2
NS = 16
NW = NC * NS
BLK = 128
CH = 16


CNT_R = 4224


def _seg_sum_body(src_hbm, dst_hbm, dstc_hbm, table_hbm, acc_hbm, cnt_hbm,
                  src_v, dst_v, dstc_v, rows0, rows1, ones_v, zc_v,
                  acc_sh, cnt_sh, g0, g1, s0, s1, *, ew, accr):
    c = lax.axis_index("c")
    s = lax.axis_index("s")
    slab = accr // NS
    cslab = CNT_R // NS
    rows_v = rows0
    bufs = (rows0, rows1)
    gsems = (g0, g1)
    ssems = (s0, s1)

    zv = jnp.zeros((16,), jnp.float32)
    ov = jnp.ones((16,), jnp.float32)

    def zrows(i, carry):
        r = i // (DH // 16)
        col = (i % (DH // 16)) * 16
        rows_v[r, pl.ds(col, 16)] = zv
        return carry
    lax.fori_loop(0, BLK * (DH // 16), zrows, 0)

    def zfill(i, carry):
        ones_v[i % BLK, pl.ds(0, 16)] = ov
        zc_v[i, pl.ds(0, 16)] = zv
        return carry
    lax.fori_loop(0, BLK, zfill, 0)

    base = s * slab
    cbase = s * cslab
    def zacc(i, carry):
        r = base + i * BLK
        nrow = jnp.minimum(slab - i * BLK, BLK)
        @pl.when(nrow == BLK)
        def _():
            pltpu.sync_copy(rows_v, acc_sh.at[pl.ds(r, BLK)])
        @pl.when(nrow < BLK)
        def _():
            pltpu.sync_copy(rows_v.at[pl.ds(0, slab % BLK)],
                            acc_sh.at[pl.ds(r, slab % BLK)])
        return carry
    lax.fori_loop(0, -(-slab // BLK), zacc, 0)
    def zcnt(i, carry):
        r = cbase + i * BLK
        nrow = jnp.minimum(cslab - i * BLK, BLK)
        @pl.when(nrow == BLK)
        def _():
            pltpu.sync_copy(zc_v, cnt_sh.at[pl.ds(r, BLK)])
        @pl.when(nrow < BLK)
        def _():
            pltpu.sync_copy(zc_v.at[pl.ds(0, cslab % BLK)],
                            cnt_sh.at[pl.ds(r, cslab % BLK)])
        return carry
    lax.fori_loop(0, -(-cslab // BLK), zcnt, 0)
    plsc.subcore_barrier()

    nblk = ew // BLK
    def chunk(t, carry):
        row0 = s * nblk + t * CH
        pltpu.sync_copy(src_hbm.at[pl.ds(c * (NS * nblk) + row0, CH)], src_v)
        pltpu.sync_copy(dst_hbm.at[pl.ds(row0, CH)], dst_v)
        pltpu.sync_copy(dstc_hbm.at[pl.ds(row0, CH)], dstc_v)
        gcp = {0: pltpu.async_copy(table_hbm.at[src_v.at[0]], bufs[0],
                                   gsems[0])}
        scp = {}
        for j in range(CH):
            if j + 1 < CH:
                if j - 1 in scp:
                    scp.pop(j - 1).wait()
                gcp[j + 1] = pltpu.async_copy(
                    table_hbm.at[src_v.at[j + 1]], bufs[(j + 1) % 2],
                    gsems[(j + 1) % 2])
            gcp.pop(j).wait()
            scp[j] = pltpu.async_copy(bufs[j % 2], acc_sh.at[dst_v.at[j]],
                                      ssems[j % 2], add=True)
            @pl.when(lax.rem(jnp.int32(j), 2) == c)
            def _():
                pltpu.sync_copy(ones_v, cnt_sh.at[dstc_v.at[j]], add=True)
        for cp in scp.values():
            cp.wait()
        return carry
    lax.fori_loop(0, nblk // CH, chunk, 0)

    plsc.subcore_barrier()
    pltpu.sync_copy(acc_sh.at[pl.ds(base, slab)],
                    acc_hbm.at[c, pl.ds(base, slab)])
    pltpu.sync_copy(cnt_sh.at[pl.ds(cbase, cslab)],
                    cnt_hbm.at[c, pl.ds(cbase, cslab)])


@functools.lru_cache(maxsize=None)
def _make_seg_kernel(ew, accr):
    mesh = plsc.VectorSubcoreMesh(core_axis_name="c", subcore_axis_name="s")
    return pl.kernel(
        functools.partial(_seg_sum_body, ew=ew, accr=accr),
        out_type=[jax.ShapeDtypeStruct((NC, accr, DH), jnp.float32),
                  jax.ShapeDtypeStruct((NC, CNT_R, 16), jnp.float32)],
        mesh=mesh,
        compiler_params=pltpu.CompilerParams(use_tc_tiling_on_sc=False),
        scratch_types=[
            pltpu.VMEM((CH, BLK), jnp.int32),
            pltpu.VMEM((CH, BLK), jnp.int32),
            pltpu.VMEM((CH, BLK), jnp.int32),
            pltpu.VMEM((BLK, DH), jnp.float32),
            pltpu.VMEM((BLK, DH), jnp.float32),
            pltpu.VMEM((BLK, 16), jnp.float32),
            pltpu.VMEM((BLK, 16), jnp.float32),
            pltpu.VMEM_SHARED((accr, DH), jnp.float32),
            pltpu.VMEM_SHARED((CNT_R, 16), jnp.float32),
            pltpu.SemaphoreType.DMA,
            pltpu.SemaphoreType.DMA,
            pltpu.SemaphoreType.DMA,
            pltpu.SemaphoreType.DMA,
        ],
    )


def _epilogue_body(acc_ref, cnt_ref, xt_ref, zr_ref, wl_ref, b_ref, wr_ref,
                   o_ref, *, act):
    ab = acc_ref[...]
    cb = cnt_ref[...]
    a = jnp.concatenate([ab[0], ab[1]], axis=-1)
    cnt = cb[0, :, 0:1] + cb[1, :, 0:1]
    agg = a / jnp.clip(cnt, 1.0, None)
    xt = xt_ref[...]
    if xt.ndim == 3:
        xt = jnp.concatenate([xt[0], xt[1]], axis=-1)
    xt = xt + zr_ref[...]
    z = (jnp.dot(agg, wl_ref[...], preferred_element_type=jnp.float32)
         + b_ref[...]
         + jnp.dot(xt, wr_ref[...], preferred_element_type=jnp.float32))
    if act == "relu":
        h = jnp.maximum(z, 0.0)
        o_ref[0] = h[:, :DH]
        o_ref[1] = h[:, DH:]
    else:
        mx = jnp.max(z, axis=-1, keepdims=True)
        e = jnp.exp(z - mx)
        o_ref[...] = z - mx - jnp.log(jnp.sum(e, axis=-1, keepdims=True))


@functools.lru_cache(maxsize=None)
def _make_epilogue(act, rows=512):
    grid = (N2 // rows,)
    if act == "relu":
        xt_spec = pl.BlockSpec((rows, D), lambda i: (i, 0))
        out_spec = pl.BlockSpec((NC, rows, DH), lambda i: (0, i, 0))
        out_shape = jax.ShapeDtypeStruct((NC, N2, DH), jnp.float32)
    else:
        xt_spec = pl.BlockSpec((NC, rows, DH), lambda i: (0, i, 0))
        out_spec = pl.BlockSpec((rows, D), lambda i: (i, 0))
        out_shape = jax.ShapeDtypeStruct((N2, D), jnp.float32)
    return pl.pallas_call(
        functools.partial(_epilogue_body, act=act),
        grid=grid,
        in_specs=[
            pl.BlockSpec((NC, rows, DH), lambda i: (0, i, 0)),
            pl.BlockSpec((NC, rows, 16), lambda i: (0, i, 0)),
            xt_spec,
            pl.BlockSpec((1, D), lambda i: (0, 0)),
            pl.BlockSpec((D, D), lambda i: (0, 0)),
            pl.BlockSpec((1, D), lambda i: (0, 0)),
            pl.BlockSpec((D, D), lambda i: (0, 0)),
        ],
        out_specs=out_spec,
        out_shape=out_shape,
    )


def _pad_edges(src, dst, garbage, ngarb):
    e = src.shape[0]
    ew = -(-e // (NS * CH * BLK)) * CH * BLK
    pad = NS * ew - e
    if pad:
        r = jnp.arange(pad, dtype=jnp.int32)
        src = jnp.concatenate([src, r % 1024])
        dst = jnp.concatenate([dst, garbage + r % ngarb])
    return src, dst, ew


def kernel(x, edge_index0, edge_index1, size0_dst, size1_dst,
           W0_l, b0_l, W0_r, W1_l, b1_l, W1_r):
    x = x.astype(jnp.float32)
    acc_r0 = N1 + 96
    acc_r1 = N2 + 128
    src0, dst0, ew0 = _pad_edges(edge_index0[0].astype(jnp.int32),
                                 edge_index0[1].astype(jnp.int32), N1, 96)
    src1, dst1, ew1 = _pad_edges(edge_index1[0].astype(jnp.int32),
                                 edge_index1[1].astype(jnp.int32), N2, 128)
    z0 = jnp.asarray(size0_dst - N1, dtype=x.dtype)
    z1 = jnp.asarray(size1_dst - N2, dtype=x.dtype)

    n0 = x.shape[0]
    x_h = jnp.concatenate([x[:, :DH], x[:, DH:]])
    src0x = jnp.concatenate([src0, src0 + n0]).reshape(-1, BLK)
    dstc0 = jnp.where(dst0 < N2, dst0, N2 + lax.rem(dst0, 128)).reshape(-1, BLK)
    dst0 = dst0.reshape(-1, BLK)
    acc0, cnt0 = _make_seg_kernel(ew0, acc_r0)(src0x, dst0, dstc0, x_h)
    z0r = jnp.full((1, D), z0, jnp.float32)
    z1r = jnp.full((1, D), z1, jnp.float32)
    h_h = _make_epilogue("relu")(acc0, cnt0, x, z0r,
                                 W0_l, b0_l.reshape(1, D), W0_r)
    src1x = jnp.concatenate([src1, src1 + N2]).reshape(-1, BLK)
    dstc1 = jnp.where(dst1 < N2, dst1, N2 + lax.rem(dst1, 128)).reshape(-1, BLK)
    dst1 = dst1.reshape(-1, BLK)
    acc1, cnt1 = _make_seg_kernel(ew1, acc_r1)(src1x, dst1, dstc1,
                                               h_h.reshape(NC * N2, DH))
    out = _make_epilogue("lsm")(acc1, cnt1, h_h, z1r,
                                W1_l, b1_l.reshape(1, D), W1_r)
    return out
